# Initial kernel scaffold; baseline (speedup 1.0000x reference)
import jax
import jax.numpy as jnp
from jax.experimental import pallas as pl


def kernel(x, edge_index, W1, b1, W2, b2):
    raise NotImplementedError("write your pallas kernel here")


# speedup vs baseline: 54.4261x; 54.4261x over previous
import functools

import jax
import jax.numpy as jnp
from jax import lax
from jax.experimental import pallas as pl
from jax.experimental.pallas import tpu as pltpu
from jax.experimental.pallas import tpu_sc as plsc

N = 10000
E = 320000
D = 16
NC, NS = 2, 16
NW = NC * NS
EPW = E // NW
CHUNK = 5000
NCHUNK = EPW // CHUNK
NPAD = 10240
DSL = NPAD // NS
RSL = NPAD // NS

_mesh = plsc.VectorSubcoreMesh(core_axis_name="c", subcore_axis_name="s")
_sc_params = pltpu.CompilerParams(use_tc_tiling_on_sc=False)


@functools.partial(
    pl.kernel,
    out_type=jax.ShapeDtypeStruct((NC * NPAD,), jnp.float32),
    mesh=_mesh,
    scratch_types=[
        pltpu.VMEM((EPW,), jnp.int32),
        pltpu.VMEM((EPW,), jnp.float32),
        pltpu.VMEM((DSL,), jnp.float32),
        pltpu.VMEM_SHARED((NPAD,), jnp.float32),
    ],
    compiler_params=_sc_params,
)
def _deg_kernel(dst_hbm, out_hbm, idx_v, ones_v, z_v, acc_sh):
    c = lax.axis_index("c")
    s = lax.axis_index("s")
    wid = s * NC + c

    def fill_ones(i, carry):
        ones_v[pl.ds(i * 16, 16)] = jnp.ones((16,), jnp.float32)
        return carry

    lax.fori_loop(0, EPW // 16, fill_ones, 0)

    def fill_zero(i, carry):
        z_v[pl.ds(i * 16, 16)] = jnp.zeros((16,), jnp.float32)
        return carry

    lax.fori_loop(0, DSL // 16, fill_zero, 0)

    pltpu.sync_copy(z_v, acc_sh.at[pl.ds(s * DSL, DSL)])
    plsc.subcore_barrier()

    pltpu.sync_copy(dst_hbm.at[pl.ds(wid * EPW, EPW)], idx_v)
    pltpu.sync_copy(ones_v, acc_sh.at[idx_v], add=True)
    plsc.subcore_barrier()

    pltpu.sync_copy(acc_sh.at[pl.ds(s * DSL, DSL)],
                    out_hbm.at[pl.ds(c * NPAD + s * DSL, DSL)])


@functools.partial(
    pl.kernel,
    out_type=jax.ShapeDtypeStruct((NC * NPAD, D), jnp.float32),
    mesh=_mesh,
    scratch_types=[
        pltpu.VMEM((CHUNK,), jnp.int32),
        pltpu.VMEM((CHUNK,), jnp.int32),
        pltpu.VMEM((CHUNK, D), jnp.float32),
        pltpu.VMEM((RSL, D), jnp.float32),
        pltpu.VMEM_SHARED((NPAD, D), jnp.float32),
        pltpu.SemaphoreType.DMA,
    ],
    compiler_params=_sc_params,
)
def _agg_kernel(y_hbm, src_hbm, dst_hbm, out_hbm,
                si_v, di_v, rows_v, z_v, acc_sh, sem):
    c = lax.axis_index("c")
    s = lax.axis_index("s")
    wid = s * NC + c

    def fill_zero(i, carry):
        z_v[i, :] = jnp.zeros((D,), jnp.float32)
        return carry

    lax.fori_loop(0, RSL, fill_zero, 0)
    pltpu.sync_copy(z_v, acc_sh.at[pl.ds(s * RSL, RSL)])
    plsc.subcore_barrier()

    for k in range(NCHUNK):
        base = wid * EPW + k * CHUNK
        pltpu.sync_copy(src_hbm.at[pl.ds(base, CHUNK)], si_v)
        pltpu.sync_copy(dst_hbm.at[pl.ds(base, CHUNK)], di_v)
        pltpu.async_copy(y_hbm.at[si_v], rows_v, sem).wait()
        pltpu.sync_copy(rows_v, acc_sh.at[di_v], add=True)

    plsc.subcore_barrier()
    pltpu.sync_copy(acc_sh.at[pl.ds(s * RSL, RSL)],
                    out_hbm.at[pl.ds(c * NPAD + s * RSL, RSL)])


def _tc1_body(x_ref, w1_ref, dp_ref, y1_ref, dinv_ref):
    deg = dp_ref[:, 0:1] + dp_ref[:, 1:2] + 1.0
    dinv = lax.rsqrt(deg)
    xw = jnp.dot(x_ref[...], w1_ref[...], preferred_element_type=jnp.float32)
    y1_ref[...] = xw * dinv
    dinv_ref[...] = dinv


_tc1 = pl.pallas_call(
    _tc1_body,
    out_shape=(jax.ShapeDtypeStruct((N, D), jnp.float32),
               jax.ShapeDtypeStruct((N, 1), jnp.float32)),
)


def _tc2_body(p0_ref, p1_ref, y1_ref, dinv_ref, b1_ref, w2_ref, y2_ref):
    agg = p0_ref[...] + p1_ref[...] + y1_ref[...]
    h = jnp.maximum(dinv_ref[...] * agg + b1_ref[...], 0.0)
    hw = jnp.dot(h, w2_ref[...], preferred_element_type=jnp.float32)
    y2_ref[...] = hw * dinv_ref[...]


_tc2 = pl.pallas_call(
    _tc2_body,
    out_shape=jax.ShapeDtypeStruct((N, D), jnp.float32),
)


def _tc3_body(q0_ref, q1_ref, y2_ref, dinv_ref, b2_ref, out_ref):
    agg = q0_ref[...] + q1_ref[...] + y2_ref[...]
    out_ref[...] = dinv_ref[...] * agg + b2_ref[...]


_tc3 = pl.pallas_call(
    _tc3_body,
    out_shape=jax.ShapeDtypeStruct((N, D), jnp.float32),
)


def kernel(x, edge_index, W1, b1, W2, b2):
    ei = edge_index.astype(jnp.int32)
    src = ei[0]
    dst = ei[1]
    degp = _deg_kernel(dst)
    dp = degp.reshape(NC, NPAD)[:, :N].T
    y1, dinv = _tc1(x, W1, dp)
    a1 = _agg_kernel(y1, src, dst)
    y2 = _tc2(a1[:N], a1[NPAD:NPAD + N], y1, dinv, b1.reshape(1, D), W2)
    a2 = _agg_kernel(y2, src, dst)
    return _tc3(a2[:N], a2[NPAD:NPAD + N], y2, dinv, b2.reshape(1, D))


# speedup vs baseline: 63.9500x; 1.1750x over previous
import functools

import jax
import jax.numpy as jnp
from jax import lax
from jax.experimental import pallas as pl
from jax.experimental.pallas import tpu as pltpu
from jax.experimental.pallas import tpu_sc as plsc

N = 10000
E = 320000
D = 16
NC, NS = 2, 16
NW = NC * NS
EPW = E // NW
CHUNK = 2000
NCHUNK = EPW // CHUNK
NPAD = 10240
DSL = NPAD // NS
RSL = NPAD // NS

_mesh = plsc.VectorSubcoreMesh(core_axis_name="c", subcore_axis_name="s")
_sc_params = pltpu.CompilerParams(use_tc_tiling_on_sc=False)


@functools.partial(
    pl.kernel,
    out_type=jax.ShapeDtypeStruct((NC * NPAD,), jnp.float32),
    mesh=_mesh,
    scratch_types=[
        pltpu.VMEM((EPW,), jnp.int32),
        pltpu.VMEM((EPW,), jnp.float32),
        pltpu.VMEM((DSL,), jnp.float32),
        pltpu.VMEM_SHARED((NPAD,), jnp.float32),
    ],
    compiler_params=_sc_params,
)
def _deg_kernel(dst_hbm, out_hbm, idx_v, ones_v, z_v, acc_sh):
    c = lax.axis_index("c")
    s = lax.axis_index("s")
    wid = s * NC + c

    def fill_ones(i, carry):
        ones_v[pl.ds(i * 16, 16)] = jnp.ones((16,), jnp.float32)
        return carry

    lax.fori_loop(0, EPW // 16, fill_ones, 0)

    def fill_zero(i, carry):
        z_v[pl.ds(i * 16, 16)] = jnp.zeros((16,), jnp.float32)
        return carry

    lax.fori_loop(0, DSL // 16, fill_zero, 0)

    pltpu.sync_copy(z_v, acc_sh.at[pl.ds(s * DSL, DSL)])
    plsc.subcore_barrier()

    pltpu.sync_copy(dst_hbm.at[pl.ds(wid * EPW, EPW)], idx_v)
    pltpu.sync_copy(ones_v, acc_sh.at[idx_v], add=True)
    plsc.subcore_barrier()

    pltpu.sync_copy(acc_sh.at[pl.ds(s * DSL, DSL)],
                    out_hbm.at[pl.ds(c * NPAD + s * DSL, DSL)])


@functools.partial(
    pl.kernel,
    out_type=jax.ShapeDtypeStruct((NC * NPAD, D), jnp.float32),
    mesh=_mesh,
    scratch_types=[
        pltpu.VMEM((2, CHUNK), jnp.int32),
        pltpu.VMEM((2, CHUNK), jnp.int32),
        pltpu.VMEM((2, CHUNK, D), jnp.float32),
        pltpu.VMEM((RSL, D), jnp.float32),
        pltpu.VMEM_SHARED((NPAD, D), jnp.float32),
        pltpu.SemaphoreType.DMA,
        pltpu.SemaphoreType.DMA,
        pltpu.SemaphoreType.DMA,
        pltpu.SemaphoreType.DMA,
    ],
    compiler_params=_sc_params,
)
def _agg_kernel(y_hbm, src_hbm, dst_hbm, out_hbm,
                si_v, di_v, rows_v, z_v, acc_sh,
                sem_i0, sem_i1, sem_g0, sem_g1):
    c = lax.axis_index("c")
    s = lax.axis_index("s")
    wid = s * NC + c
    sem_i = (sem_i0, sem_i1)
    sem_g = (sem_g0, sem_g1)

    def start_idx(k):
        b = k % 2
        base = wid * EPW + k * CHUNK
        pltpu.async_copy(src_hbm.at[pl.ds(base, CHUNK)], si_v.at[b], sem_i[b])
        pltpu.async_copy(dst_hbm.at[pl.ds(base, CHUNK)], di_v.at[b], sem_i[b])

    def start_gather(k):
        b = k % 2
        return pltpu.async_copy(y_hbm.at[si_v.at[b]], rows_v.at[b], sem_g[b])

    def fill_zero(i, carry):
        z_v[i, :] = jnp.zeros((D,), jnp.float32)
        return carry

    start_idx(0)
    start_idx(1)
    lax.fori_loop(0, RSL, fill_zero, 0)
    pltpu.sync_copy(z_v, acc_sh.at[pl.ds(s * RSL, RSL)])
    plsc.subcore_barrier()

    gathers = [None, None]
    pltpu.make_async_copy(src_hbm.at[pl.ds(0, CHUNK)], si_v.at[0], sem_i[0]).wait()
    pltpu.make_async_copy(dst_hbm.at[pl.ds(0, CHUNK)], di_v.at[0], sem_i[0]).wait()
    gathers[0] = start_gather(0)

    for k in range(NCHUNK):
        b = k % 2
        nb = (k + 1) % 2
        if k + 1 < NCHUNK:
            pltpu.make_async_copy(src_hbm.at[pl.ds(0, CHUNK)],
                                  si_v.at[nb], sem_i[nb]).wait()
            pltpu.make_async_copy(dst_hbm.at[pl.ds(0, CHUNK)],
                                  di_v.at[nb], sem_i[nb]).wait()
        gathers[b].wait()
        if k + 1 < NCHUNK:
            gathers[nb] = start_gather(k + 1)
        pltpu.sync_copy(rows_v.at[b], acc_sh.at[di_v.at[b]], add=True)
        if k + 2 < NCHUNK:
            start_idx(k + 2)

    plsc.subcore_barrier()
    pltpu.sync_copy(acc_sh.at[pl.ds(s * RSL, RSL)],
                    out_hbm.at[pl.ds(c * NPAD + s * RSL, RSL)])


def _tc1_body(x_ref, w1_ref, dp_ref, y1_ref, dinv_ref):
    deg = dp_ref[:, 0:1] + dp_ref[:, 1:2] + 1.0
    dinv = lax.rsqrt(deg)
    xw = jnp.dot(x_ref[...], w1_ref[...], preferred_element_type=jnp.float32)
    y1_ref[...] = xw * dinv
    dinv_ref[...] = dinv


_tc1 = pl.pallas_call(
    _tc1_body,
    out_shape=(jax.ShapeDtypeStruct((N, D), jnp.float32),
               jax.ShapeDtypeStruct((N, 1), jnp.float32)),
)


def _tc2_body(a_ref, y1_ref, dinv_ref, b1_ref, w2_ref, y2_ref):
    agg = a_ref[0:N, :] + a_ref[NPAD:NPAD + N, :] + y1_ref[...]
    h = jnp.maximum(dinv_ref[...] * agg + b1_ref[...], 0.0)
    hw = jnp.dot(h, w2_ref[...], preferred_element_type=jnp.float32)
    y2_ref[...] = hw * dinv_ref[...]


_tc2 = pl.pallas_call(
    _tc2_body,
    out_shape=jax.ShapeDtypeStruct((N, D), jnp.float32),
)


def _tc3_body(a_ref, y2_ref, dinv_ref, b2_ref, out_ref):
    agg = a_ref[0:N, :] + a_ref[NPAD:NPAD + N, :] + y2_ref[...]
    out_ref[...] = dinv_ref[...] * agg + b2_ref[...]


_tc3 = pl.pallas_call(
    _tc3_body,
    out_shape=jax.ShapeDtypeStruct((N, D), jnp.float32),
)


def kernel(x, edge_index, W1, b1, W2, b2):
    ei = edge_index.astype(jnp.int32)
    src = ei[0]
    dst = ei[1]
    degp = _deg_kernel(dst)
    dp = degp.reshape(NC, NPAD)[:, :N].T
    y1, dinv = _tc1(x, W1, dp)
    a1 = _agg_kernel(y1, src, dst)
    y2 = _tc2(a1, y1, dinv, b1.reshape(1, D), W2)
    a2 = _agg_kernel(y2, src, dst)
    return _tc3(a2, y2, dinv, b2.reshape(1, D))


# speedup vs baseline: 69.1268x; 1.0810x over previous
import functools

import jax
import jax.numpy as jnp
from jax import lax
from jax.experimental import pallas as pl
from jax.experimental.pallas import tpu as pltpu
from jax.experimental.pallas import tpu_sc as plsc

N = 10000
E = 320000
D = 16
NC, NS = 2, 16
NW = NC * NS
EPW = E // NW
CHUNK = 2000
NCHUNK = EPW // CHUNK
NPAD = 10240
DSL = NPAD // NS
RSL = NPAD // NS

_mesh = plsc.VectorSubcoreMesh(core_axis_name="c", subcore_axis_name="s")
_sc_params = pltpu.CompilerParams(use_tc_tiling_on_sc=False)


@functools.partial(
    pl.kernel,
    out_type=jax.ShapeDtypeStruct((NC * NPAD,), jnp.float32),
    mesh=_mesh,
    scratch_types=[
        pltpu.VMEM((EPW,), jnp.int32),
        pltpu.VMEM((EPW,), jnp.float32),
        pltpu.VMEM((DSL,), jnp.float32),
        pltpu.VMEM_SHARED((NPAD,), jnp.float32),
    ],
    compiler_params=_sc_params,
)
def _deg_kernel(ei_hbm, out_hbm, idx_v, ones_v, z_v, acc_sh):
    c = lax.axis_index("c")
    s = lax.axis_index("s")
    wid = s * NC + c

    def fill_ones(i, carry):
        ones_v[pl.ds(i * 16, 16)] = jnp.ones((16,), jnp.float32)
        return carry

    lax.fori_loop(0, EPW // 16, fill_ones, 0)

    def fill_zero(i, carry):
        z_v[pl.ds(i * 16, 16)] = jnp.zeros((16,), jnp.float32)
        return carry

    lax.fori_loop(0, DSL // 16, fill_zero, 0)

    pltpu.sync_copy(z_v, acc_sh.at[pl.ds(s * DSL, DSL)])
    plsc.subcore_barrier()

    pltpu.sync_copy(ei_hbm.at[pl.ds(E + wid * EPW, EPW)], idx_v)
    pltpu.sync_copy(ones_v, acc_sh.at[idx_v], add=True)
    plsc.subcore_barrier()

    pltpu.sync_copy(acc_sh.at[pl.ds(s * DSL, DSL)],
                    out_hbm.at[pl.ds(c * NPAD + s * DSL, DSL)])


@functools.partial(
    pl.kernel,
    out_type=jax.ShapeDtypeStruct((NC * NPAD, D), jnp.float32),
    mesh=_mesh,
    scratch_types=[
        pltpu.VMEM((2, CHUNK), jnp.int32),
        pltpu.VMEM((2, CHUNK), jnp.int32),
        pltpu.VMEM((2, CHUNK, D), jnp.float32),
        pltpu.VMEM((RSL, D), jnp.float32),
        pltpu.VMEM_SHARED((NPAD, D), jnp.float32),
        pltpu.SemaphoreType.DMA,
        pltpu.SemaphoreType.DMA,
        pltpu.SemaphoreType.DMA,
        pltpu.SemaphoreType.DMA,
    ],
    compiler_params=_sc_params,
)
def _agg_kernel(y_hbm, ei_hbm, out_hbm,
                si_v, di_v, rows_v, z_v, acc_sh,
                sem_i0, sem_i1, sem_g0, sem_g1):
    c = lax.axis_index("c")
    s = lax.axis_index("s")
    wid = s * NC + c
    sem_i = (sem_i0, sem_i1)
    sem_g = (sem_g0, sem_g1)

    def start_idx(k):
        b = k % 2
        base = wid * EPW + k * CHUNK
        pltpu.async_copy(ei_hbm.at[pl.ds(base, CHUNK)], si_v.at[b], sem_i[b])
        pltpu.async_copy(ei_hbm.at[pl.ds(E + base, CHUNK)], di_v.at[b], sem_i[b])

    def start_gather(k):
        b = k % 2
        return pltpu.async_copy(y_hbm.at[si_v.at[b]], rows_v.at[b], sem_g[b])

    def fill_zero(i, carry):
        z_v[i, :] = jnp.zeros((D,), jnp.float32)
        return carry

    start_idx(0)
    start_idx(1)
    lax.fori_loop(0, RSL, fill_zero, 0)
    pltpu.sync_copy(z_v, acc_sh.at[pl.ds(s * RSL, RSL)])
    plsc.subcore_barrier()

    gathers = [None, None]
    pltpu.make_async_copy(ei_hbm.at[pl.ds(0, CHUNK)], si_v.at[0], sem_i[0]).wait()
    pltpu.make_async_copy(ei_hbm.at[pl.ds(0, CHUNK)], di_v.at[0], sem_i[0]).wait()
    gathers[0] = start_gather(0)

    for k in range(NCHUNK):
        b = k % 2
        nb = (k + 1) % 2
        if k + 1 < NCHUNK:
            pltpu.make_async_copy(ei_hbm.at[pl.ds(0, CHUNK)],
                                  si_v.at[nb], sem_i[nb]).wait()
            pltpu.make_async_copy(ei_hbm.at[pl.ds(0, CHUNK)],
                                  di_v.at[nb], sem_i[nb]).wait()
        gathers[b].wait()
        if k + 1 < NCHUNK:
            gathers[nb] = start_gather(k + 1)
        pltpu.sync_copy(rows_v.at[b], acc_sh.at[di_v.at[b]], add=True)
        if k + 2 < NCHUNK:
            start_idx(k + 2)

    plsc.subcore_barrier()
    pltpu.sync_copy(acc_sh.at[pl.ds(s * RSL, RSL)],
                    out_hbm.at[pl.ds(c * NPAD + s * RSL, RSL)])


def _tc1_body(x_ref, w1_ref, dp_ref, y1_ref, dinv_ref):
    deg = dp_ref[:, 0:1] + dp_ref[:, 1:2] + 1.0
    dinv = lax.rsqrt(deg)
    xw = jnp.dot(x_ref[...], w1_ref[...], preferred_element_type=jnp.float32)
    y1_ref[...] = xw * dinv
    dinv_ref[...] = dinv


_tc1 = pl.pallas_call(
    _tc1_body,
    out_shape=(jax.ShapeDtypeStruct((N, D), jnp.float32),
               jax.ShapeDtypeStruct((N, 1), jnp.float32)),
)


def _tc2_body(a_ref, y1_ref, dinv_ref, b1_ref, w2_ref, y2_ref):
    agg = a_ref[0:N, :] + a_ref[NPAD:NPAD + N, :] + y1_ref[...]
    h = jnp.maximum(dinv_ref[...] * agg + b1_ref[...], 0.0)
    hw = jnp.dot(h, w2_ref[...], preferred_element_type=jnp.float32)
    y2_ref[...] = hw * dinv_ref[...]


_tc2 = pl.pallas_call(
    _tc2_body,
    out_shape=jax.ShapeDtypeStruct((N, D), jnp.float32),
)


def _tc3_body(a_ref, y2_ref, dinv_ref, b2_ref, out_ref):
    agg = a_ref[0:N, :] + a_ref[NPAD:NPAD + N, :] + y2_ref[...]
    out_ref[...] = dinv_ref[...] * agg + b2_ref[...]


_tc3 = pl.pallas_call(
    _tc3_body,
    out_shape=jax.ShapeDtypeStruct((N, D), jnp.float32),
)


def kernel(x, edge_index, W1, b1, W2, b2):
    ei = edge_index.astype(jnp.int32).reshape(2 * E)
    degp = _deg_kernel(ei)
    dp = degp.reshape(NC, NPAD)[:, :N].T
    y1, dinv = _tc1(x, W1, dp)
    a1 = _agg_kernel(y1, ei)
    y2 = _tc2(a1, y1, dinv, b1.reshape(1, D), W2)
    a2 = _agg_kernel(y2, ei)
    return _tc3(a2, y2, dinv, b2.reshape(1, D))


# speedup vs baseline: 85.9710x; 1.2437x over previous
import functools

import jax
import jax.numpy as jnp
from jax import lax
from jax.experimental import pallas as pl
from jax.experimental.pallas import tpu as pltpu
from jax.experimental.pallas import tpu_sc as plsc

N = 10000
E = 320000
D = 16
NC, NS = 2, 16
NW = NC * NS
EPW = E // NW
CHUNK = 2000
NCHUNK = EPW // CHUNK
NPAD = 10240
DSL = NPAD // NS
RSL = NPAD // NS
NP8 = N // 8
NPAD8 = NPAD // 8
RSL8 = RSL // 8

_mesh = plsc.VectorSubcoreMesh(core_axis_name="c", subcore_axis_name="s")
_sc_params = pltpu.CompilerParams(use_tc_tiling_on_sc=False)


@functools.partial(
    pl.kernel,
    out_type=jax.ShapeDtypeStruct((NC * NPAD,), jnp.float32),
    mesh=_mesh,
    scratch_types=[
        pltpu.VMEM((EPW,), jnp.int32),
        pltpu.VMEM((EPW,), jnp.float32),
        pltpu.VMEM((DSL,), jnp.float32),
        pltpu.VMEM_SHARED((NPAD,), jnp.float32),
    ],
    compiler_params=_sc_params,
)
def _deg_kernel(ei_hbm, out_hbm, idx_v, ones_v, z_v, acc_sh):
    c = lax.axis_index("c")
    s = lax.axis_index("s")
    wid = s * NC + c

    def fill_ones(i, carry):
        ones_v[pl.ds(i * 16, 16)] = jnp.ones((16,), jnp.float32)
        return carry

    lax.fori_loop(0, EPW // 16, fill_ones, 0)

    def fill_zero(i, carry):
        z_v[pl.ds(i * 16, 16)] = jnp.zeros((16,), jnp.float32)
        return carry

    lax.fori_loop(0, DSL // 16, fill_zero, 0)

    pltpu.sync_copy(z_v, acc_sh.at[pl.ds(s * DSL, DSL)])
    plsc.subcore_barrier()

    pltpu.sync_copy(ei_hbm.at[pl.ds(E + wid * EPW, EPW)], idx_v)
    pltpu.sync_copy(ones_v, acc_sh.at[idx_v], add=True)
    plsc.subcore_barrier()

    pltpu.sync_copy(acc_sh.at[pl.ds(s * DSL, DSL)],
                    out_hbm.at[pl.ds(c * NPAD + s * DSL, DSL)])


@functools.partial(
    pl.kernel,
    out_type=jax.ShapeDtypeStruct((NC * NPAD, D), jnp.float32),
    mesh=_mesh,
    scratch_types=[
        pltpu.VMEM((2, CHUNK), jnp.int32),
        pltpu.VMEM((2, CHUNK), jnp.int32),
        pltpu.VMEM((2, CHUNK, D), jnp.float32),
        pltpu.VMEM((RSL, D), jnp.float32),
        pltpu.VMEM_SHARED((NPAD, D), jnp.float32),
        pltpu.SemaphoreType.DMA,
        pltpu.SemaphoreType.DMA,
        pltpu.SemaphoreType.DMA,
        pltpu.SemaphoreType.DMA,
    ],
    compiler_params=_sc_params,
)
def _agg_kernel(y_hbm, ei_hbm, out_hbm,
                si_v, di_v, rows_v, z_v, acc_sh,
                sem_i0, sem_i1, sem_g0, sem_g1):
    c = lax.axis_index("c")
    s = lax.axis_index("s")
    wid = s * NC + c
    sem_i = (sem_i0, sem_i1)
    sem_g = (sem_g0, sem_g1)

    def start_idx(k):
        b = k % 2
        base = wid * EPW + k * CHUNK
        pltpu.async_copy(ei_hbm.at[pl.ds(base, CHUNK)], si_v.at[b], sem_i[b])
        pltpu.async_copy(ei_hbm.at[pl.ds(E + base, CHUNK)], di_v.at[b], sem_i[b])

    def start_gather(k):
        b = k % 2
        return pltpu.async_copy(y_hbm.at[si_v.at[b]], rows_v.at[b], sem_g[b])

    def fill_zero(i, carry):
        z_v[i, :] = jnp.zeros((D,), jnp.float32)
        return carry

    start_idx(0)
    start_idx(1)
    lax.fori_loop(0, RSL, fill_zero, 0)
    pltpu.sync_copy(z_v, acc_sh.at[pl.ds(s * RSL, RSL)])
    plsc.subcore_barrier()

    gathers = [None, None]
    pltpu.make_async_copy(ei_hbm.at[pl.ds(0, CHUNK)], si_v.at[0], sem_i[0]).wait()
    pltpu.make_async_copy(ei_hbm.at[pl.ds(0, CHUNK)], di_v.at[0], sem_i[0]).wait()
    gathers[0] = start_gather(0)

    for k in range(NCHUNK):
        b = k % 2
        nb = (k + 1) % 2
        if k + 1 < NCHUNK:
            pltpu.make_async_copy(ei_hbm.at[pl.ds(0, CHUNK)],
                                  si_v.at[nb], sem_i[nb]).wait()
            pltpu.make_async_copy(ei_hbm.at[pl.ds(0, CHUNK)],
                                  di_v.at[nb], sem_i[nb]).wait()
        gathers[b].wait()
        if k + 1 < NCHUNK:
            gathers[nb] = start_gather(k + 1)
        pltpu.sync_copy(rows_v.at[b], acc_sh.at[di_v.at[b]], add=True)
        if k + 2 < NCHUNK:
            start_idx(k + 2)

    plsc.subcore_barrier()
    pltpu.sync_copy(acc_sh.at[pl.ds(s * RSL, RSL)],
                    out_hbm.at[pl.ds(c * NPAD + s * RSL, RSL)])


def _tc1_body(x3_ref, w1_ref, dp3_ref, y1_ref, db_ref):
    deg = dp3_ref[:, :, 0] + dp3_ref[:, :, 1] + 1.0
    dinv = lax.rsqrt(deg)
    cols_y, cols_d = [], []
    for j in range(8):
        dj = dinv[:, j:j + 1]
        xw = jnp.dot(x3_ref[:, j, :], w1_ref[...],
                     preferred_element_type=jnp.float32)
        cols_y.append(xw * dj)
        cols_d.append(jnp.broadcast_to(dj, (NP8, D)))
    y1_ref[...] = jnp.concatenate(cols_y, axis=1)
    db_ref[...] = jnp.concatenate(cols_d, axis=1)


_tc1 = pl.pallas_call(
    _tc1_body,
    out_shape=(jax.ShapeDtypeStruct((NP8, 128), jnp.float32),
               jax.ShapeDtypeStruct((NP8, 128), jnp.float32)),
)


def _tc2_body(a_ref, y1_ref, db_ref, b1_ref, w2_ref, y2_ref):
    agg = a_ref[0:NP8, :] + a_ref[NPAD8:NPAD8 + NP8, :] + y1_ref[...]
    h = jnp.maximum(db_ref[...] * agg + b1_ref[...], 0.0)
    outs = [jnp.dot(h[:, 16 * j:16 * j + 16], w2_ref[...],
                    preferred_element_type=jnp.float32) for j in range(8)]
    y2_ref[...] = jnp.concatenate(outs, axis=1) * db_ref[...]


_tc2 = pl.pallas_call(
    _tc2_body,
    out_shape=jax.ShapeDtypeStruct((NP8, 128), jnp.float32),
)


def _tc3_body(a_ref, y2_ref, db_ref, b2_ref, out_ref):
    agg = a_ref[0:NP8, :] + a_ref[NPAD8:NPAD8 + NP8, :] + y2_ref[...]
    out_ref[...] = db_ref[...] * agg + b2_ref[...]


_tc3 = pl.pallas_call(
    _tc3_body,
    out_shape=jax.ShapeDtypeStruct((NP8, 128), jnp.float32),
)


def kernel(x, edge_index, W1, b1, W2, b2):
    ei = edge_index.astype(jnp.int32).reshape(2 * E)
    b1p = jnp.tile(b1.reshape(1, D), (1, 8))
    b2p = jnp.tile(b2.reshape(1, D), (1, 8))
    degp = _deg_kernel(ei)
    dp3 = degp.reshape(NC, NPAD)[:, :N].T.reshape(NP8, 8, NC)
    y1, db = _tc1(x.reshape(NP8, 8, 128), W1, dp3)
    a1 = _agg_kernel(y1.reshape(N, D), ei)
    y2 = _tc2(a1.reshape(NC * NPAD8, 128), y1, db, b1p, W2)
    a2 = _agg_kernel(y2.reshape(N, D), ei)
    return _tc3(a2.reshape(NC * NPAD8, 128), y2, db, b2p).reshape(N, D)


# speedup vs baseline: 94.7085x; 1.1016x over previous
import functools

import jax
import jax.numpy as jnp
from jax import lax
from jax.experimental import pallas as pl
from jax.experimental.pallas import tpu as pltpu
from jax.experimental.pallas import tpu_sc as plsc

N = 10000
E = 320000
D = 16
NC, NS = 2, 16
NW = NC * NS
EPW = E // NW
CHUNK = 2000
NCHUNK = EPW // CHUNK
NPAD = 10240
DSL = NPAD // NS
RSL = NPAD // NS
NP8 = N // 8
NPAD8 = NPAD // 8
RSL8 = RSL // 8

_mesh = plsc.VectorSubcoreMesh(core_axis_name="c", subcore_axis_name="s")
_sc_params = pltpu.CompilerParams(use_tc_tiling_on_sc=False)


@functools.partial(
    pl.kernel,
    out_type=jax.ShapeDtypeStruct((NC * NPAD,), jnp.float32),
    mesh=_mesh,
    scratch_types=[
        pltpu.VMEM((EPW,), jnp.int32),
        pltpu.VMEM((EPW,), jnp.float32),
        pltpu.VMEM((DSL,), jnp.float32),
        pltpu.VMEM_SHARED((NPAD,), jnp.float32),
    ],
    compiler_params=_sc_params,
)
def _deg_kernel(ei_hbm, out_hbm, idx_v, ones_v, z_v, acc_sh):
    c = lax.axis_index("c")
    s = lax.axis_index("s")
    wid = s * NC + c

    def fill_ones(i, carry):
        ones_v[pl.ds(i * 16, 16)] = jnp.ones((16,), jnp.float32)
        return carry

    lax.fori_loop(0, EPW // 16, fill_ones, 0)

    def fill_zero(i, carry):
        z_v[pl.ds(i * 16, 16)] = jnp.zeros((16,), jnp.float32)
        return carry

    lax.fori_loop(0, DSL // 16, fill_zero, 0)

    pltpu.sync_copy(z_v, acc_sh.at[pl.ds(s * DSL, DSL)])
    plsc.subcore_barrier()

    pltpu.sync_copy(ei_hbm.at[pl.ds(E + wid * EPW, EPW)], idx_v)
    pltpu.sync_copy(ones_v, acc_sh.at[idx_v], add=True)
    plsc.subcore_barrier()

    pltpu.sync_copy(acc_sh.at[pl.ds(s * DSL, DSL)],
                    out_hbm.at[pl.ds(c * NPAD + s * DSL, DSL)])


@functools.partial(
    pl.kernel,
    out_type=jax.ShapeDtypeStruct((NC * NPAD, D), jnp.float32),
    mesh=_mesh,
    scratch_types=[
        pltpu.VMEM((2, CHUNK), jnp.int32),
        pltpu.VMEM((2, CHUNK), jnp.int32),
        pltpu.VMEM((2, CHUNK, D), jnp.float32),
        pltpu.VMEM((RSL, D), jnp.float32),
        pltpu.VMEM_SHARED((NPAD, D), jnp.float32),
        pltpu.SemaphoreType.DMA,
        pltpu.SemaphoreType.DMA,
        pltpu.SemaphoreType.DMA,
        pltpu.SemaphoreType.DMA,
    ],
    compiler_params=_sc_params,
)
def _agg_kernel(y_hbm, ei_hbm, out_hbm,
                si_v, di_v, rows_v, z_v, acc_sh,
                sem_i0, sem_i1, sem_g0, sem_g1):
    c = lax.axis_index("c")
    s = lax.axis_index("s")
    wid = s * NC + c
    sem_i = (sem_i0, sem_i1)
    sem_g = (sem_g0, sem_g1)

    def start_idx(k):
        b = k % 2
        base = wid * EPW + k * CHUNK
        pltpu.async_copy(ei_hbm.at[pl.ds(base, CHUNK)], si_v.at[b], sem_i[b])
        pltpu.async_copy(ei_hbm.at[pl.ds(E + base, CHUNK)], di_v.at[b], sem_i[b])

    def start_gather(k):
        b = k % 2
        return pltpu.async_copy(y_hbm.at[si_v.at[b]], rows_v.at[b], sem_g[b])

    def fill_zero(i, carry):
        z_v[i, :] = jnp.zeros((D,), jnp.float32)
        return carry

    start_idx(0)
    start_idx(1)
    lax.fori_loop(0, RSL, fill_zero, 0)
    pltpu.sync_copy(z_v, acc_sh.at[pl.ds(s * RSL, RSL)])
    plsc.subcore_barrier()

    gathers = [None, None]
    pltpu.make_async_copy(ei_hbm.at[pl.ds(0, CHUNK)], si_v.at[0], sem_i[0]).wait()
    pltpu.make_async_copy(ei_hbm.at[pl.ds(0, CHUNK)], di_v.at[0], sem_i[0]).wait()
    gathers[0] = start_gather(0)

    for k in range(NCHUNK):
        b = k % 2
        nb = (k + 1) % 2
        if k + 1 < NCHUNK:
            pltpu.make_async_copy(ei_hbm.at[pl.ds(0, CHUNK)],
                                  si_v.at[nb], sem_i[nb]).wait()
            pltpu.make_async_copy(ei_hbm.at[pl.ds(0, CHUNK)],
                                  di_v.at[nb], sem_i[nb]).wait()
        gathers[b].wait()
        if k + 1 < NCHUNK:
            gathers[nb] = start_gather(k + 1)
        pltpu.sync_copy(rows_v.at[b], acc_sh.at[di_v.at[b]], add=True)
        if k + 2 < NCHUNK:
            start_idx(k + 2)

    plsc.subcore_barrier()
    pltpu.sync_copy(acc_sh.at[pl.ds(s * RSL, RSL)],
                    out_hbm.at[pl.ds(c * NPAD + s * RSL, RSL)])


def _tc1_body(x3_ref, w1b_ref, dp3_ref, y1_ref, db_ref):
    deg = dp3_ref[:, :, 0] + dp3_ref[:, :, 1] + 1.0
    dinv = lax.rsqrt(deg)
    ci = lax.broadcasted_iota(jnp.int32, (8, 128), 1) // D
    ri = lax.broadcasted_iota(jnp.int32, (8, 128), 0)
    sel = jnp.where(ci == ri, 1.0, 0.0).astype(jnp.float32)
    db = jnp.dot(dinv, sel, preferred_element_type=jnp.float32)
    yp = jnp.dot(x3_ref[:, 0, :], w1b_ref[0:128, :],
                 preferred_element_type=jnp.float32)
    for j in range(1, 8):
        yp = yp + jnp.dot(x3_ref[:, j, :], w1b_ref[128 * j:128 * j + 128, :],
                          preferred_element_type=jnp.float32)
    y1_ref[...] = yp * db
    db_ref[...] = db


_tc1 = pl.pallas_call(
    _tc1_body,
    out_shape=(jax.ShapeDtypeStruct((NP8, 128), jnp.float32),
               jax.ShapeDtypeStruct((NP8, 128), jnp.float32)),
)


def _tc2_body(a_ref, y1_ref, db_ref, b1_ref, w2b_ref, y2_ref):
    agg = a_ref[0:NP8, :] + a_ref[NPAD8:NPAD8 + NP8, :] + y1_ref[...]
    h = jnp.maximum(db_ref[...] * agg + b1_ref[...], 0.0)
    hw = jnp.dot(h, w2b_ref[...], preferred_element_type=jnp.float32)
    y2_ref[...] = hw * db_ref[...]


_tc2 = pl.pallas_call(
    _tc2_body,
    out_shape=jax.ShapeDtypeStruct((NP8, 128), jnp.float32),
)


def _tc3_body(a_ref, y2_ref, db_ref, b2_ref, out_ref):
    agg = a_ref[0:NP8, :] + a_ref[NPAD8:NPAD8 + NP8, :] + y2_ref[...]
    out_ref[...] = db_ref[...] * agg + b2_ref[...]


_tc3 = pl.pallas_call(
    _tc3_body,
    out_shape=jax.ShapeDtypeStruct((NP8, 128), jnp.float32),
)


def kernel(x, edge_index, W1, b1, W2, b2):
    ei = edge_index.astype(jnp.int32).reshape(2 * E)
    b1p = jnp.tile(b1.reshape(1, D), (1, 8))
    b2p = jnp.tile(b2.reshape(1, D), (1, 8))
    eye8 = jnp.eye(8, dtype=jnp.float32)
    w1b = jnp.kron(eye8, W1)
    w2b = jnp.kron(eye8, W2)
    degp = _deg_kernel(ei)
    dp3 = degp.reshape(NC, NPAD)[:, :N].T.reshape(NP8, 8, NC)
    y1, db = _tc1(x.reshape(NP8, 8, 128), w1b, dp3)
    a1 = _agg_kernel(y1.reshape(N, D), ei)
    y2 = _tc2(a1.reshape(NC * NPAD8, 128), y1, db, b1p, w2b)
    a2 = _agg_kernel(y2.reshape(N, D), ei)
    return _tc3(a2.reshape(NC * NPAD8, 128), y2, db, b2p).reshape(N, D)


# speedup vs baseline: 97.7219x; 1.0318x over previous
import functools

import jax
import jax.numpy as jnp
from jax import lax
from jax.experimental import pallas as pl
from jax.experimental.pallas import tpu as pltpu
from jax.experimental.pallas import tpu_sc as plsc

N = 10000
E = 320000
D = 16
NC, NS = 2, 16
NW = NC * NS
EPW = E // NW
CHUNK = 2000
NCHUNK = EPW // CHUNK
NPAD = 10240
DSL = NPAD // NS
RSL = NPAD // NS
NP8 = N // 8
NPAD8 = NPAD // 8
RSL8 = RSL // 8

_mesh = plsc.VectorSubcoreMesh(core_axis_name="c", subcore_axis_name="s")
_sc_params = pltpu.CompilerParams(use_tc_tiling_on_sc=False)


@functools.partial(
    pl.kernel,
    out_type=jax.ShapeDtypeStruct((NC * NPAD,), jnp.float32),
    mesh=_mesh,
    scratch_types=[
        pltpu.VMEM((EPW,), jnp.int32),
        pltpu.VMEM((EPW,), jnp.float32),
        pltpu.VMEM((DSL,), jnp.float32),
        pltpu.VMEM_SHARED((NPAD,), jnp.float32),
    ],
    compiler_params=_sc_params,
)
def _deg_kernel(ei_hbm, out_hbm, idx_v, ones_v, z_v, acc_sh):
    c = lax.axis_index("c")
    s = lax.axis_index("s")
    wid = s * NC + c

    def fill_ones(i, carry):
        ones_v[pl.ds(i * 16, 16)] = jnp.ones((16,), jnp.float32)
        return carry

    lax.fori_loop(0, EPW // 16, fill_ones, 0)

    def fill_zero(i, carry):
        z_v[pl.ds(i * 16, 16)] = jnp.zeros((16,), jnp.float32)
        return carry

    lax.fori_loop(0, DSL // 16, fill_zero, 0)

    pltpu.sync_copy(z_v, acc_sh.at[pl.ds(s * DSL, DSL)])
    plsc.subcore_barrier()

    pltpu.sync_copy(ei_hbm.at[pl.ds(E + wid * EPW, EPW)], idx_v)
    pltpu.sync_copy(ones_v, acc_sh.at[idx_v], add=True)
    plsc.subcore_barrier()

    pltpu.sync_copy(acc_sh.at[pl.ds(s * DSL, DSL)],
                    out_hbm.at[pl.ds(c * NPAD + s * DSL, DSL)])


@functools.partial(
    pl.kernel,
    out_type=jax.ShapeDtypeStruct((NC * NPAD, D), jnp.float32),
    mesh=_mesh,
    scratch_types=[
        pltpu.VMEM((2, CHUNK), jnp.int32),
        pltpu.VMEM((2, CHUNK), jnp.int32),
        pltpu.VMEM((2, CHUNK, D), jnp.float32),
        pltpu.VMEM((RSL, D), jnp.float32),
        pltpu.VMEM_SHARED((NPAD, D), jnp.float32),
        pltpu.SemaphoreType.DMA,
        pltpu.SemaphoreType.DMA,
        pltpu.SemaphoreType.DMA,
        pltpu.SemaphoreType.DMA,
    ],
    compiler_params=_sc_params,
)
def _agg_kernel(y_hbm, ei_hbm, out_hbm,
                si_v, di_v, rows_v, z_v, acc_sh,
                sem_i0, sem_i1, sem_g0, sem_g1):
    c = lax.axis_index("c")
    s = lax.axis_index("s")
    wid = s * NC + c
    sem_i = (sem_i0, sem_i1)
    sem_g = (sem_g0, sem_g1)

    def start_idx(k):
        b = k % 2
        base = wid * EPW + k * CHUNK
        pltpu.async_copy(ei_hbm.at[pl.ds(base, CHUNK)], si_v.at[b], sem_i[b])
        pltpu.async_copy(ei_hbm.at[pl.ds(E + base, CHUNK)], di_v.at[b], sem_i[b])

    def start_gather(k):
        b = k % 2
        return pltpu.async_copy(y_hbm.at[si_v.at[b]], rows_v.at[b], sem_g[b])

    def fill_zero(i, carry):
        z_v[i, :] = jnp.zeros((D,), jnp.float32)
        return carry

    start_idx(0)
    start_idx(1)
    lax.fori_loop(0, RSL, fill_zero, 0)
    pltpu.sync_copy(z_v, acc_sh.at[pl.ds(s * RSL, RSL)])
    plsc.subcore_barrier()

    gathers = [None, None]
    pltpu.make_async_copy(ei_hbm.at[pl.ds(0, CHUNK)], si_v.at[0], sem_i[0]).wait()
    pltpu.make_async_copy(ei_hbm.at[pl.ds(0, CHUNK)], di_v.at[0], sem_i[0]).wait()
    gathers[0] = start_gather(0)

    for k in range(NCHUNK):
        b = k % 2
        nb = (k + 1) % 2
        if k + 1 < NCHUNK:
            pltpu.make_async_copy(ei_hbm.at[pl.ds(0, CHUNK)],
                                  si_v.at[nb], sem_i[nb]).wait()
            pltpu.make_async_copy(ei_hbm.at[pl.ds(0, CHUNK)],
                                  di_v.at[nb], sem_i[nb]).wait()
        gathers[b].wait()
        if k + 1 < NCHUNK:
            gathers[nb] = start_gather(k + 1)
        pltpu.sync_copy(rows_v.at[b], acc_sh.at[di_v.at[b]], add=True)
        if k + 2 < NCHUNK:
            start_idx(k + 2)

    plsc.subcore_barrier()
    pltpu.sync_copy(acc_sh.at[pl.ds(s * RSL, RSL)],
                    out_hbm.at[pl.ds(c * NPAD + s * RSL, RSL)])


def _tc1a_body(x3_ref, w1b_ref, xwp_ref):
    yp = jnp.dot(x3_ref[:, 0, :], w1b_ref[0:128, :],
                 preferred_element_type=jnp.float32)
    for j in range(1, 8):
        yp = yp + jnp.dot(x3_ref[:, j, :], w1b_ref[128 * j:128 * j + 128, :],
                          preferred_element_type=jnp.float32)
    xwp_ref[...] = yp


_tc1a = pl.pallas_call(
    _tc1a_body,
    out_shape=jax.ShapeDtypeStruct((NP8, 128), jnp.float32),
)


def _tc1b_body(xwp_ref, dp3_ref, y1_ref, db_ref):
    deg = dp3_ref[:, :, 0] + dp3_ref[:, :, 1] + 1.0
    dinv = lax.rsqrt(deg)
    ci = lax.broadcasted_iota(jnp.int32, (8, 128), 1) // D
    ri = lax.broadcasted_iota(jnp.int32, (8, 128), 0)
    sel = jnp.where(ci == ri, 1.0, 0.0).astype(jnp.float32)
    db = jnp.dot(dinv, sel, preferred_element_type=jnp.float32)
    y1_ref[...] = xwp_ref[...] * db
    db_ref[...] = db


_tc1b = pl.pallas_call(
    _tc1b_body,
    out_shape=(jax.ShapeDtypeStruct((NP8, 128), jnp.float32),
               jax.ShapeDtypeStruct((NP8, 128), jnp.float32)),
)


def _tc2_body(a_ref, y1_ref, db_ref, b1_ref, w2b_ref, y2_ref):
    agg = a_ref[0:NP8, :] + a_ref[NPAD8:NPAD8 + NP8, :] + y1_ref[...]
    h = jnp.maximum(db_ref[...] * agg + b1_ref[...], 0.0)
    hw = jnp.dot(h, w2b_ref[...], preferred_element_type=jnp.float32)
    y2_ref[...] = hw * db_ref[...]


_tc2 = pl.pallas_call(
    _tc2_body,
    out_shape=jax.ShapeDtypeStruct((NP8, 128), jnp.float32),
)


def _tc3_body(a_ref, y2_ref, db_ref, b2_ref, out_ref):
    agg = a_ref[0:NP8, :] + a_ref[NPAD8:NPAD8 + NP8, :] + y2_ref[...]
    out_ref[...] = db_ref[...] * agg + b2_ref[...]


_tc3 = pl.pallas_call(
    _tc3_body,
    out_shape=jax.ShapeDtypeStruct((NP8, 128), jnp.float32),
)


def kernel(x, edge_index, W1, b1, W2, b2):
    ei = edge_index.astype(jnp.int32).reshape(2 * E)
    b1p = jnp.tile(b1.reshape(1, D), (1, 8))
    b2p = jnp.tile(b2.reshape(1, D), (1, 8))
    eye8 = jnp.eye(8, dtype=jnp.float32)
    w1b = jnp.kron(eye8, W1)
    w2b = jnp.kron(eye8, W2)
    degp = _deg_kernel(ei)
    xwp = _tc1a(x.reshape(NP8, 8, 128), w1b)
    dp3 = degp.reshape(NC, NPAD)[:, :N].T.reshape(NP8, 8, NC)
    y1, db = _tc1b(xwp, dp3)
    a1 = _agg_kernel(y1.reshape(N, D), ei)
    y2 = _tc2(a1.reshape(NC * NPAD8, 128), y1, db, b1p, w2b)
    a2 = _agg_kernel(y2.reshape(N, D), ei)
    return _tc3(a2.reshape(NC * NPAD8, 128), y2, db, b2p).reshape(N, D)


# speedup vs baseline: 98.0866x; 1.0037x over previous
import functools

import jax
import jax.numpy as jnp
from jax import lax
from jax.experimental import pallas as pl
from jax.experimental.pallas import tpu as pltpu
from jax.experimental.pallas import tpu_sc as plsc

N = 10000
E = 320000
D = 16
NC, NS = 2, 16
NW = NC * NS
EPW = E // NW
CHUNK = 2000
NCHUNK = EPW // CHUNK
NPAD = 10240
DSL = NPAD // NS
RSL = NPAD // NS
NP8 = N // 8
NPAD8 = NPAD // 8
RSL8 = RSL // 8

_mesh = plsc.VectorSubcoreMesh(core_axis_name="c", subcore_axis_name="s")
_sc_params = pltpu.CompilerParams(use_tc_tiling_on_sc=False)


@functools.partial(
    pl.kernel,
    out_type=jax.ShapeDtypeStruct((NC * NPAD,), jnp.float32),
    mesh=_mesh,
    scratch_types=[
        pltpu.VMEM((EPW,), jnp.int32),
        pltpu.VMEM((EPW,), jnp.float32),
        pltpu.VMEM((DSL,), jnp.float32),
        pltpu.VMEM_SHARED((NPAD,), jnp.float32),
    ],
    compiler_params=_sc_params,
)
def _deg_kernel(ei_hbm, out_hbm, idx_v, ones_v, z_v, acc_sh):
    c = lax.axis_index("c")
    s = lax.axis_index("s")
    wid = s * NC + c

    def fill_ones(i, carry):
        ones_v[pl.ds(i * 16, 16)] = jnp.ones((16,), jnp.float32)
        return carry

    lax.fori_loop(0, EPW // 16, fill_ones, 0)

    def fill_zero(i, carry):
        z_v[pl.ds(i * 16, 16)] = jnp.zeros((16,), jnp.float32)
        return carry

    lax.fori_loop(0, DSL // 16, fill_zero, 0)

    pltpu.sync_copy(z_v, acc_sh.at[pl.ds(s * DSL, DSL)])
    plsc.subcore_barrier()

    pltpu.sync_copy(ei_hbm.at[pl.ds(E + wid * EPW, EPW)], idx_v)
    pltpu.sync_copy(ones_v, acc_sh.at[idx_v], add=True)
    plsc.subcore_barrier()

    pltpu.sync_copy(acc_sh.at[pl.ds(s * DSL, DSL)],
                    out_hbm.at[pl.ds(c * NPAD + s * DSL, DSL)])


@functools.partial(
    pl.kernel,
    out_type=jax.ShapeDtypeStruct((NC * NPAD, D), jnp.float32),
    mesh=_mesh,
    scratch_types=[
        pltpu.VMEM((2, CHUNK), jnp.int32),
        pltpu.VMEM((2, CHUNK), jnp.int32),
        pltpu.VMEM((2, CHUNK, D), jnp.float32),
        pltpu.VMEM((RSL, D), jnp.float32),
        pltpu.VMEM_SHARED((NPAD, D), jnp.float32),
        pltpu.SemaphoreType.DMA,
        pltpu.SemaphoreType.DMA,
        pltpu.SemaphoreType.DMA,
        pltpu.SemaphoreType.DMA,
    ],
    compiler_params=_sc_params,
)
def _agg_kernel(y_hbm, ei_hbm, out_hbm,
                si_v, di_v, rows_v, z_v, acc_sh,
                sem_i0, sem_i1, sem_g0, sem_g1):
    c = lax.axis_index("c")
    s = lax.axis_index("s")
    wid = s * NC + c
    sem_i = (sem_i0, sem_i1)
    sem_g = (sem_g0, sem_g1)

    def start_idx(k):
        b = k % 2
        base = wid * EPW + k * CHUNK
        pltpu.async_copy(ei_hbm.at[pl.ds(base, CHUNK)], si_v.at[b], sem_i[b])
        pltpu.async_copy(ei_hbm.at[pl.ds(E + base, CHUNK)], di_v.at[b], sem_i[b])

    def start_gather(k):
        b = k % 2
        return pltpu.async_copy(y_hbm.at[si_v.at[b]], rows_v.at[b], sem_g[b])

    def fill_zero(i, carry):
        z_v[i, :] = jnp.zeros((D,), jnp.float32)
        return carry

    start_idx(0)
    start_idx(1)
    lax.fori_loop(0, RSL, fill_zero, 0)
    pltpu.sync_copy(z_v, acc_sh.at[pl.ds(s * RSL, RSL)])
    plsc.subcore_barrier()

    gathers = [None, None]
    pltpu.make_async_copy(ei_hbm.at[pl.ds(0, CHUNK)], si_v.at[0], sem_i[0]).wait()
    pltpu.make_async_copy(ei_hbm.at[pl.ds(0, CHUNK)], di_v.at[0], sem_i[0]).wait()
    gathers[0] = start_gather(0)

    for k in range(NCHUNK):
        b = k % 2
        nb = (k + 1) % 2
        if k + 1 < NCHUNK:
            pltpu.make_async_copy(ei_hbm.at[pl.ds(0, CHUNK)],
                                  si_v.at[nb], sem_i[nb]).wait()
            pltpu.make_async_copy(ei_hbm.at[pl.ds(0, CHUNK)],
                                  di_v.at[nb], sem_i[nb]).wait()
        gathers[b].wait()
        if k + 1 < NCHUNK:
            gathers[nb] = start_gather(k + 1)
        pltpu.sync_copy(rows_v.at[b], acc_sh.at[di_v.at[b]], add=True)
        if k + 2 < NCHUNK:
            start_idx(k + 2)

    plsc.subcore_barrier()
    pltpu.sync_copy(acc_sh.at[pl.ds(s * RSL, RSL)],
                    out_hbm.at[pl.ds(c * NPAD + s * RSL, RSL)])


def _tc1a_body(x3_ref, w1b_ref, xwp_ref):
    yp = jnp.dot(x3_ref[:, 0, :], w1b_ref[0:128, :],
                 preferred_element_type=jnp.float32)
    for j in range(1, 8):
        yp = yp + jnp.dot(x3_ref[:, j, :], w1b_ref[128 * j:128 * j + 128, :],
                          preferred_element_type=jnp.float32)
    xwp_ref[...] = yp


_tc1a = pl.pallas_call(
    _tc1a_body,
    out_shape=jax.ShapeDtypeStruct((NP8, 128), jnp.float32),
)


def _tc1b_body(xwp_ref, dp3_ref, sel_ref, y1_ref, db_ref):
    deg = dp3_ref[:, :, 0] + dp3_ref[:, :, 1] + 1.0
    degb = jnp.dot(deg, sel_ref[...], preferred_element_type=jnp.float32)
    db = lax.rsqrt(degb)
    y1_ref[...] = xwp_ref[...] * db
    db_ref[...] = db


_tc1b = pl.pallas_call(
    _tc1b_body,
    out_shape=(jax.ShapeDtypeStruct((NP8, 128), jnp.float32),
               jax.ShapeDtypeStruct((NP8, 128), jnp.float32)),
)


def _tc2_body(a_ref, y1_ref, db_ref, b1_ref, w2b_ref, y2_ref):
    agg = a_ref[0:NP8, :] + a_ref[NPAD8:NPAD8 + NP8, :] + y1_ref[...]
    h = jnp.maximum(db_ref[...] * agg + b1_ref[...], 0.0)
    hw = jnp.dot(h, w2b_ref[...], preferred_element_type=jnp.float32)
    y2_ref[...] = hw * db_ref[...]


_tc2 = pl.pallas_call(
    _tc2_body,
    out_shape=jax.ShapeDtypeStruct((NP8, 128), jnp.float32),
)


def _tc3_body(a_ref, y2_ref, db_ref, b2_ref, out_ref):
    agg = a_ref[0:NP8, :] + a_ref[NPAD8:NPAD8 + NP8, :] + y2_ref[...]
    out_ref[...] = db_ref[...] * agg + b2_ref[...]


_tc3 = pl.pallas_call(
    _tc3_body,
    out_shape=jax.ShapeDtypeStruct((NP8, 128), jnp.float32),
)


def kernel(x, edge_index, W1, b1, W2, b2):
    ei = edge_index.astype(jnp.int32).reshape(2 * E)
    b1p = jnp.tile(b1.reshape(1, D), (1, 8))
    b2p = jnp.tile(b2.reshape(1, D), (1, 8))
    eye8 = jnp.eye(8, dtype=jnp.float32)
    w1b = jnp.kron(eye8, W1)
    w2b = jnp.kron(eye8, W2)
    degp = _deg_kernel(ei)
    xwp = _tc1a(x.reshape(NP8, 8, 128), w1b)
    dp3 = degp.reshape(NC, NPAD)[:, :N].T.reshape(NP8, 8, NC)
    sel = jnp.kron(eye8, jnp.ones((1, D), jnp.float32))
    y1, db = _tc1b(xwp, dp3, sel)
    a1 = _agg_kernel(y1.reshape(N, D), ei)
    y2 = _tc2(a1.reshape(NC * NPAD8, 128), y1, db, b1p, w2b)
    a2 = _agg_kernel(y2.reshape(N, D), ei)
    return _tc3(a2.reshape(NC * NPAD8, 128), y2, db, b2p).reshape(N, D)


# speedup vs baseline: 98.9867x; 1.0092x over previous
import functools

import jax
import jax.numpy as jnp
from jax import lax
from jax.experimental import pallas as pl
from jax.experimental.pallas import tpu as pltpu
from jax.experimental.pallas import tpu_sc as plsc

N = 10000
E = 320000
D = 16
NC, NS = 2, 16
NW = NC * NS
EPW = E // NW
CHUNK = 2000
NCHUNK = EPW // CHUNK
NPAD = 10240
DSL = NPAD // NS
RSL = NPAD // NS
NP8 = N // 8
NPAD8 = NPAD // 8
RSL8 = RSL // 8

_mesh = plsc.VectorSubcoreMesh(core_axis_name="c", subcore_axis_name="s")
_sc_params = pltpu.CompilerParams(use_tc_tiling_on_sc=False)


@functools.partial(
    pl.kernel,
    out_type=jax.ShapeDtypeStruct((NC * NPAD,), jnp.float32),
    mesh=_mesh,
    scratch_types=[
        pltpu.VMEM((EPW,), jnp.int32),
        pltpu.VMEM((EPW,), jnp.float32),
        pltpu.VMEM((DSL,), jnp.float32),
        pltpu.VMEM_SHARED((NPAD,), jnp.float32),
        pltpu.SemaphoreType.DMA,
    ],
    compiler_params=_sc_params,
)
def _deg_kernel(ei_hbm, out_hbm, idx_v, ones_v, z_v, acc_sh, sem):
    c = lax.axis_index("c")
    s = lax.axis_index("s")
    wid = s * NC + c
    idx_cp = pltpu.async_copy(ei_hbm.at[pl.ds(E + wid * EPW, EPW)], idx_v, sem)

    def fill_ones(i, carry):
        ones_v[pl.ds(i * 16, 16)] = jnp.ones((16,), jnp.float32)
        return carry

    lax.fori_loop(0, EPW // 16, fill_ones, 0)

    def fill_zero(i, carry):
        z_v[pl.ds(i * 16, 16)] = jnp.zeros((16,), jnp.float32)
        return carry

    lax.fori_loop(0, DSL // 16, fill_zero, 0)

    pltpu.sync_copy(z_v, acc_sh.at[pl.ds(s * DSL, DSL)])
    plsc.subcore_barrier()

    idx_cp.wait()
    pltpu.sync_copy(ones_v, acc_sh.at[idx_v], add=True)
    plsc.subcore_barrier()

    pltpu.sync_copy(acc_sh.at[pl.ds(s * DSL, DSL)],
                    out_hbm.at[pl.ds(c * NPAD + s * DSL, DSL)])


@functools.partial(
    pl.kernel,
    out_type=jax.ShapeDtypeStruct((NC * NPAD, D), jnp.float32),
    mesh=_mesh,
    scratch_types=[
        pltpu.VMEM((2, CHUNK), jnp.int32),
        pltpu.VMEM((2, CHUNK), jnp.int32),
        pltpu.VMEM((2, CHUNK, D), jnp.float32),
        pltpu.VMEM((RSL, D), jnp.float32),
        pltpu.VMEM_SHARED((NPAD, D), jnp.float32),
        pltpu.SemaphoreType.DMA,
        pltpu.SemaphoreType.DMA,
        pltpu.SemaphoreType.DMA,
        pltpu.SemaphoreType.DMA,
    ],
    compiler_params=_sc_params,
)
def _agg_kernel(y_hbm, ei_hbm, out_hbm,
                si_v, di_v, rows_v, z_v, acc_sh,
                sem_i0, sem_i1, sem_g0, sem_g1):
    c = lax.axis_index("c")
    s = lax.axis_index("s")
    wid = s * NC + c
    sem_i = (sem_i0, sem_i1)
    sem_g = (sem_g0, sem_g1)

    def start_idx(k):
        b = k % 2
        base = wid * EPW + k * CHUNK
        pltpu.async_copy(ei_hbm.at[pl.ds(base, CHUNK)], si_v.at[b], sem_i[b])
        pltpu.async_copy(ei_hbm.at[pl.ds(E + base, CHUNK)], di_v.at[b], sem_i[b])

    def start_gather(k):
        b = k % 2
        return pltpu.async_copy(y_hbm.at[si_v.at[b]], rows_v.at[b], sem_g[b])

    def fill_zero(i, carry):
        z_v[i, :] = jnp.zeros((D,), jnp.float32)
        return carry

    start_idx(0)
    start_idx(1)
    lax.fori_loop(0, RSL, fill_zero, 0)
    pltpu.sync_copy(z_v, acc_sh.at[pl.ds(s * RSL, RSL)])
    plsc.subcore_barrier()

    gathers = [None, None]
    pltpu.make_async_copy(ei_hbm.at[pl.ds(0, CHUNK)], si_v.at[0], sem_i[0]).wait()
    pltpu.make_async_copy(ei_hbm.at[pl.ds(0, CHUNK)], di_v.at[0], sem_i[0]).wait()
    gathers[0] = start_gather(0)

    for k in range(NCHUNK):
        b = k % 2
        nb = (k + 1) % 2
        if k + 1 < NCHUNK:
            pltpu.make_async_copy(ei_hbm.at[pl.ds(0, CHUNK)],
                                  si_v.at[nb], sem_i[nb]).wait()
            pltpu.make_async_copy(ei_hbm.at[pl.ds(0, CHUNK)],
                                  di_v.at[nb], sem_i[nb]).wait()
        gathers[b].wait()
        if k + 1 < NCHUNK:
            gathers[nb] = start_gather(k + 1)
        pltpu.sync_copy(rows_v.at[b], acc_sh.at[di_v.at[b]], add=True)
        if k + 2 < NCHUNK:
            start_idx(k + 2)

    plsc.subcore_barrier()
    pltpu.sync_copy(acc_sh.at[pl.ds(s * RSL, RSL)],
                    out_hbm.at[pl.ds(c * NPAD + s * RSL, RSL)])


def _tc1a_body(x3_ref, w1b_ref, xwp_ref):
    yp = jnp.dot(x3_ref[:, 0, :], w1b_ref[0:128, :],
                 preferred_element_type=jnp.float32)
    for j in range(1, 8):
        yp = yp + jnp.dot(x3_ref[:, j, :], w1b_ref[128 * j:128 * j + 128, :],
                          preferred_element_type=jnp.float32)
    xwp_ref[...] = yp


_tc1a = pl.pallas_call(
    _tc1a_body,
    out_shape=jax.ShapeDtypeStruct((NP8, 128), jnp.float32),
)


def _tc1b_body(xwp_ref, dp3_ref, sel_ref, y1_ref, db_ref):
    deg = dp3_ref[:, :, 0] + dp3_ref[:, :, 1] + 1.0
    degb = jnp.dot(deg, sel_ref[...], preferred_element_type=jnp.float32)
    db = lax.rsqrt(degb)
    y1_ref[...] = xwp_ref[...] * db
    db_ref[...] = db


_tc1b = pl.pallas_call(
    _tc1b_body,
    out_shape=(jax.ShapeDtypeStruct((NP8, 128), jnp.float32),
               jax.ShapeDtypeStruct((NP8, 128), jnp.float32)),
)


def _tc2_body(a_ref, y1_ref, db_ref, b1_ref, w2b_ref, y2_ref):
    agg = a_ref[0:NP8, :] + a_ref[NPAD8:NPAD8 + NP8, :] + y1_ref[...]
    h = jnp.maximum(db_ref[...] * agg + b1_ref[...], 0.0)
    hw = jnp.dot(h, w2b_ref[...], preferred_element_type=jnp.float32)
    y2_ref[...] = hw * db_ref[...]


_tc2 = pl.pallas_call(
    _tc2_body,
    out_shape=jax.ShapeDtypeStruct((NP8, 128), jnp.float32),
)


def _tc3_body(a_ref, y2_ref, db_ref, b2_ref, out_ref):
    agg = a_ref[0:NP8, :] + a_ref[NPAD8:NPAD8 + NP8, :] + y2_ref[...]
    out_ref[...] = db_ref[...] * agg + b2_ref[...]


_tc3 = pl.pallas_call(
    _tc3_body,
    out_shape=jax.ShapeDtypeStruct((NP8, 128), jnp.float32),
)


def kernel(x, edge_index, W1, b1, W2, b2):
    ei = edge_index.astype(jnp.int32).reshape(2 * E)
    b1p = jnp.tile(b1.reshape(1, D), (1, 8))
    b2p = jnp.tile(b2.reshape(1, D), (1, 8))
    eye8 = jnp.eye(8, dtype=jnp.float32)
    w1b = jnp.kron(eye8, W1)
    w2b = jnp.kron(eye8, W2)
    degp = _deg_kernel(ei)
    xwp = _tc1a(x.reshape(NP8, 8, 128), w1b)
    dp3 = degp.reshape(NC, NPAD)[:, :N].T.reshape(NP8, 8, NC)
    sel = jnp.kron(eye8, jnp.ones((1, D), jnp.float32))
    y1, db = _tc1b(xwp, dp3, sel)
    a1 = _agg_kernel(y1.reshape(N, D), ei)
    y2 = _tc2(a1.reshape(NC * NPAD8, 128), y1, db, b1p, w2b)
    a2 = _agg_kernel(y2.reshape(N, D), ei)
    return _tc3(a2.reshape(NC * NPAD8, 128), y2, db, b2p).reshape(N, D)


# speedup vs baseline: 99.2255x; 1.0024x over previous
import functools

import jax
import jax.numpy as jnp
from jax import lax
from jax.experimental import pallas as pl
from jax.experimental.pallas import tpu as pltpu
from jax.experimental.pallas import tpu_sc as plsc

N = 10000
E = 320000
D = 16
NC, NS = 2, 16
NW = NC * NS
EPW = E // NW
CHUNK = 2000
NCHUNK = EPW // CHUNK
NPAD = 10240
DSL = NPAD // NS
RSL = NPAD // NS
NP8 = N // 8
NPAD8 = NPAD // 8
RSL8 = RSL // 8

_mesh = plsc.VectorSubcoreMesh(core_axis_name="c", subcore_axis_name="s")
_sc_params = pltpu.CompilerParams(use_tc_tiling_on_sc=False)


@functools.partial(
    pl.kernel,
    out_type=jax.ShapeDtypeStruct((NC * NPAD,), jnp.float32),
    mesh=_mesh,
    scratch_types=[
        pltpu.VMEM((EPW,), jnp.int32),
        pltpu.VMEM((EPW,), jnp.float32),
        pltpu.VMEM((DSL,), jnp.float32),
        pltpu.VMEM_SHARED((NPAD,), jnp.float32),
        pltpu.SemaphoreType.DMA,
    ],
    compiler_params=_sc_params,
)
def _deg_kernel(ei_hbm, out_hbm, idx_v, ones_v, z_v, acc_sh, sem):
    c = lax.axis_index("c")
    s = lax.axis_index("s")
    wid = s * NC + c
    idx_cp = pltpu.async_copy(ei_hbm.at[pl.ds(E + wid * EPW, EPW)], idx_v, sem)

    def fill_ones(i, carry):
        ones_v[pl.ds(i * 16, 16)] = jnp.ones((16,), jnp.float32)
        return carry

    lax.fori_loop(0, EPW // 16, fill_ones, 0)

    def fill_zero(i, carry):
        z_v[pl.ds(i * 16, 16)] = jnp.zeros((16,), jnp.float32)
        return carry

    lax.fori_loop(0, DSL // 16, fill_zero, 0)

    pltpu.sync_copy(z_v, acc_sh.at[pl.ds(s * DSL, DSL)])
    plsc.subcore_barrier()

    idx_cp.wait()
    pltpu.sync_copy(ones_v, acc_sh.at[idx_v], add=True)
    plsc.subcore_barrier()

    pltpu.sync_copy(acc_sh.at[pl.ds(s * DSL, DSL)],
                    out_hbm.at[pl.ds(c * NPAD + s * DSL, DSL)])


@functools.partial(
    pl.kernel,
    out_type=jax.ShapeDtypeStruct((NC * NPAD, D), jnp.float32),
    mesh=_mesh,
    scratch_types=[
        pltpu.VMEM((3, CHUNK), jnp.int32),
        pltpu.VMEM((3, CHUNK), jnp.int32),
        pltpu.VMEM((2, CHUNK, D), jnp.float32),
        pltpu.VMEM((RSL, D), jnp.float32),
        pltpu.VMEM_SHARED((NPAD, D), jnp.float32),
        pltpu.SemaphoreType.DMA,
        pltpu.SemaphoreType.DMA,
        pltpu.SemaphoreType.DMA,
        pltpu.SemaphoreType.DMA,
        pltpu.SemaphoreType.DMA,
        pltpu.SemaphoreType.DMA,
        pltpu.SemaphoreType.DMA,
    ],
    compiler_params=_sc_params,
)
def _agg_kernel(y_hbm, ei_hbm, out_hbm,
                si_v, di_v, rows_v, z_v, acc_sh,
                sem_i0, sem_i1, sem_i2, sem_g0, sem_g1, sem_s0, sem_s1):
    c = lax.axis_index("c")
    s = lax.axis_index("s")
    wid = s * NC + c
    sem_i = (sem_i0, sem_i1, sem_i2)
    sem_g = (sem_g0, sem_g1)
    sem_s = (sem_s0, sem_s1)

    def start_idx(k):
        b = k % 3
        base = wid * EPW + k * CHUNK
        pltpu.async_copy(ei_hbm.at[pl.ds(base, CHUNK)], si_v.at[b], sem_i[b])
        pltpu.async_copy(ei_hbm.at[pl.ds(E + base, CHUNK)], di_v.at[b], sem_i[b])

    def wait_idx(k):
        b = k % 3
        pltpu.make_async_copy(ei_hbm.at[pl.ds(0, CHUNK)], si_v.at[b],
                              sem_i[b]).wait()
        pltpu.make_async_copy(ei_hbm.at[pl.ds(0, CHUNK)], di_v.at[b],
                              sem_i[b]).wait()

    def start_gather(k):
        return pltpu.async_copy(y_hbm.at[si_v.at[k % 3]], rows_v.at[k % 2],
                                sem_g[k % 2])

    def start_scatter(k):
        return pltpu.async_copy(rows_v.at[k % 2], acc_sh.at[di_v.at[k % 3]],
                                sem_s[k % 2], add=True)

    def fill_zero(i, carry):
        z_v[i, :] = jnp.zeros((D,), jnp.float32)
        return carry

    start_idx(0)
    start_idx(1)
    start_idx(2)
    lax.fori_loop(0, RSL, fill_zero, 0)
    pltpu.sync_copy(z_v, acc_sh.at[pl.ds(s * RSL, RSL)])
    plsc.subcore_barrier()

    gath = [None, None]
    scat = [None, None]
    wait_idx(0)
    gath[0] = start_gather(0)

    for k in range(NCHUNK):
        b = k % 2
        nb = (k + 1) % 2
        if k >= 1:
            scat[nb].wait()
            if k + 2 < NCHUNK:
                start_idx(k + 2)
        gath[b].wait()
        if k + 1 < NCHUNK:
            wait_idx(k + 1)
            gath[nb] = start_gather(k + 1)
        scat[b] = start_scatter(k)

    scat[(NCHUNK - 1) % 2].wait()
    plsc.subcore_barrier()
    pltpu.sync_copy(acc_sh.at[pl.ds(s * RSL, RSL)],
                    out_hbm.at[pl.ds(c * NPAD + s * RSL, RSL)])


def _tc1a_body(x3_ref, w1b_ref, xwp_ref):
    yp = jnp.dot(x3_ref[:, 0, :], w1b_ref[0:128, :],
                 preferred_element_type=jnp.float32)
    for j in range(1, 8):
        yp = yp + jnp.dot(x3_ref[:, j, :], w1b_ref[128 * j:128 * j + 128, :],
                          preferred_element_type=jnp.float32)
    xwp_ref[...] = yp


_tc1a = pl.pallas_call(
    _tc1a_body,
    out_shape=jax.ShapeDtypeStruct((NP8, 128), jnp.float32),
)


def _tc1b_body(xwp_ref, dp3_ref, sel_ref, y1_ref, db_ref):
    deg = dp3_ref[:, :, 0] + dp3_ref[:, :, 1] + 1.0
    degb = jnp.dot(deg, sel_ref[...], preferred_element_type=jnp.float32)
    db = lax.rsqrt(degb)
    y1_ref[...] = xwp_ref[...] * db
    db_ref[...] = db


_tc1b = pl.pallas_call(
    _tc1b_body,
    out_shape=(jax.ShapeDtypeStruct((NP8, 128), jnp.float32),
               jax.ShapeDtypeStruct((NP8, 128), jnp.float32)),
)


def _tc2_body(a_ref, y1_ref, db_ref, b1_ref, w2b_ref, y2_ref):
    agg = a_ref[0:NP8, :] + a_ref[NPAD8:NPAD8 + NP8, :] + y1_ref[...]
    h = jnp.maximum(db_ref[...] * agg + b1_ref[...], 0.0)
    hw = jnp.dot(h, w2b_ref[...], preferred_element_type=jnp.float32)
    y2_ref[...] = hw * db_ref[...]


_tc2 = pl.pallas_call(
    _tc2_body,
    out_shape=jax.ShapeDtypeStruct((NP8, 128), jnp.float32),
)


def _tc3_body(a_ref, y2_ref, db_ref, b2_ref, out_ref):
    agg = a_ref[0:NP8, :] + a_ref[NPAD8:NPAD8 + NP8, :] + y2_ref[...]
    out_ref[...] = db_ref[...] * agg + b2_ref[...]


_tc3 = pl.pallas_call(
    _tc3_body,
    out_shape=jax.ShapeDtypeStruct((NP8, 128), jnp.float32),
)


def kernel(x, edge_index, W1, b1, W2, b2):
    ei = edge_index.astype(jnp.int32).reshape(2 * E)
    b1p = jnp.tile(b1.reshape(1, D), (1, 8))
    b2p = jnp.tile(b2.reshape(1, D), (1, 8))
    eye8 = jnp.eye(8, dtype=jnp.float32)
    w1b = jnp.kron(eye8, W1)
    w2b = jnp.kron(eye8, W2)
    degp = _deg_kernel(ei)
    xwp = _tc1a(x.reshape(NP8, 8, 128), w1b)
    dp3 = degp.reshape(NC, NPAD)[:, :N].T.reshape(NP8, 8, NC)
    sel = jnp.kron(eye8, jnp.ones((1, D), jnp.float32))
    y1, db = _tc1b(xwp, dp3, sel)
    a1 = _agg_kernel(y1.reshape(N, D), ei)
    y2 = _tc2(a1.reshape(NC * NPAD8, 128), y1, db, b1p, w2b)
    a2 = _agg_kernel(y2.reshape(N, D), ei)
    return _tc3(a2.reshape(NC * NPAD8, 128), y2, db, b2p).reshape(N, D)
